# baseline (speedup 1.0000x reference)
import functools

import jax
import jax.numpy as jnp
from jax import lax
from jax.experimental import pallas as pl
from jax.experimental.pallas import tpu as pltpu
from jax.experimental.pallas import tpu_sc as plsc

_NC = 2
_NS = 16
_NW = _NC * _NS
_CHUNK = 128

_NJ = 26
_NB = 4096
_D = 32
_BPW = _NB // _NW * _NJ


@functools.lru_cache(maxsize=None)
def _make_gather():
    n_chunks = _BPW // _CHUNK
    mesh = plsc.VectorSubcoreMesh(core_axis_name="c", subcore_axis_name="s")

    @functools.partial(
        pl.kernel,
        mesh=mesh,
        out_type=jax.ShapeDtypeStruct((_NJ, _D // 8, _NW, 8, 128), jnp.float32),
        scratch_types=[
            pltpu.VMEM((_BPW,), jnp.int32),
            pltpu.VMEM((_BPW, _D), jnp.float32),
            pltpu.VMEM((2, _D // 8, 8, 128), jnp.float32),
            pltpu.SemaphoreType.DMA,
            pltpu.SemaphoreType.DMA,
            pltpu.SemaphoreType.DMA,
        ],
        compiler_params=pltpu.CompilerParams(
            use_tc_tiling_on_sc=False, needs_layout_passes=False
        ),
    )
    def gather_kernel(table_hbm, idx_hbm, out_hbm, idx_v, rows_v, buf_v,
                      sem_g, sem_o0, sem_o1):
        wid = lax.axis_index("s") * _NC + lax.axis_index("c")
        base = wid * _BPW
        pltpu.sync_copy(idx_hbm.at[pl.ds(base, _BPW)], idx_v)

        def fire(i, carry):
            off = i * _CHUNK
            pltpu.async_copy(
                table_hbm.at[idx_v.at[pl.ds(off, _CHUNK)]],
                rows_v.at[pl.ds(off, _CHUNK)],
                sem_g,
            )
            return carry

        lax.fori_loop(0, n_chunks, fire, 0)
        pltpu.make_async_copy(
            table_hbm.at[pl.ds(0, _BPW)], rows_v, sem_g
        ).wait()

        sems = (sem_o0, sem_o1)
        iota26 = lax.iota(jnp.int32, 16) * _NJ

        def transpose_j(j, p, drain):
            sem = sems[p]
            if drain:
                for d_blk in range(_D // 8):
                    pltpu.make_async_copy(
                        buf_v.at[p, d_blk],
                        out_hbm.at[0, d_blk, wid],
                        sem,
                    ).wait()
            for k in range(8):
                row_vec = iota26 + (k * 16 * _NJ + j)
                for d in range(_D):
                    col_vec = jnp.full((16,), d, jnp.int32)
                    vals = plsc.load_gather(rows_v, [row_vec, col_vec])
                    buf_v[p, d // 8, d % 8, pl.ds(k * 16, 16)] = vals
            for d_blk in range(_D // 8):
                pltpu.async_copy(
                    buf_v.at[p, d_blk], out_hbm.at[j, d_blk, wid], sem
                )

        transpose_j(0, 0, False)
        transpose_j(1, 1, False)

        def body2(i, carry):
            j = i * 2
            transpose_j(j, 0, True)
            transpose_j(j + 1, 1, True)
            return carry

        lax.fori_loop(1, _NJ // 2, body2, 0)
        for p in range(2):
            for d_blk in range(_D // 8):
                pltpu.make_async_copy(
                    buf_v.at[p, d_blk], out_hbm.at[0, d_blk, wid], sems[p]
                ).wait()

    return gather_kernel


def kernel(x, weight):
    idx = x.reshape(-1).astype(jnp.int32)
    out5 = _make_gather()(weight, idx)
    return out5.transpose(2, 4, 0, 1, 3).reshape(_NB, _NJ, _D)
